# baseline (speedup 1.0000x reference)
import functools
import jax
import jax.numpy as jnp
from jax import lax
from jax.experimental import pallas as pl
from jax.experimental.pallas import tpu as pltpu
from jax.experimental.pallas import tpu_sc as plsc

E = 160000
NC = 88000
NCYC = 16000

BR = 1000


def _relu(x):
    return jnp.maximum(x, 0.0)


def _full(shape):
    return pl.BlockSpec(shape, lambda i: (0,) * len(shape))


def _rows(br, off=0):
    return pl.BlockSpec((br, 128), lambda i, o=off: (o + i, 0))


def _tc1_body(e2c1, e2c2, b1, b2, ca, bc,
              w20, bb20, w21, bb21, w22, bb22,
              w10, bb10, w11, bb11,
              we0, bbe0, we1, bbe1,
              eps_c,
              cycle_out, lac_out):
    x = jnp.concatenate([e2c2[...], b2[...], e2c1[...], b1[...]], axis=1)
    h = _relu(jnp.dot(x, w20[...], preferred_element_type=jnp.float32) + bb20[...])
    h = _relu(jnp.dot(h, w21[...], preferred_element_type=jnp.float32) + bb21[...])
    lift = jnp.dot(h, w22[...], preferred_element_type=jnp.float32) + bb22[...]

    s = 1.0 + eps_c[0, 0]
    cin = s * jnp.concatenate([ca[...], bc[...]], axis=1) + lift
    h = _relu(jnp.dot(cin, w10[...], preferred_element_type=jnp.float32) + bb10[...])
    cycle_out[...] = jnp.dot(h, w11[...], preferred_element_type=jnp.float32) + bb11[...]

    ein = jnp.concatenate([lift, ca[...]], axis=1)
    h = _relu(jnp.dot(ein, we0[...], preferred_element_type=jnp.float32) + bbe0[...])
    lac_out[...] = jnp.dot(h, we1[...], preferred_element_type=jnp.float32) + bbe1[...]


def _tc1(e2c1, e2c2, b3, ca, params):
    cm2 = params["cycle_mlp_2"]
    cm1 = params["cycle_mlp_1"]
    em1 = params["edge_mlp_1"]
    wargs = [cm2[0][0], cm2[0][1], cm2[1][0], cm2[1][1], cm2[2][0], cm2[2][1],
             cm1[0][0], cm1[0][1], cm1[1][0], cm1[1][1],
             em1[0][0], em1[0][1], em1[1][0], em1[1][1],
             params["eps_cycle_1"]]
    wspecs = [_full(w.shape) for w in wargs]
    nb = NC // BR
    return pl.pallas_call(
        _tc1_body,
        grid=(nb,),
        in_specs=[_rows(BR), _rows(BR), _rows(BR, 0), _rows(BR, nb),
                  _rows(BR), _rows(BR, 2 * nb)] + wspecs,
        out_specs=[_rows(BR), _rows(BR)],
        out_shape=[jax.ShapeDtypeStruct((NC, 128), jnp.float32),
                   jax.ShapeDtypeStruct((NC, 128), jnp.float32)],
    )(e2c1, e2c2, b3, b3, ca, b3, *wargs)


def _tc2_body(lac, blac, wa1, wb1, wa2, wb2, y1, y2):
    y1[...] = (jnp.dot(lac[...], wa1[...], preferred_element_type=jnp.float32)
               + jnp.dot(blac[...], wb1[...], preferred_element_type=jnp.float32))
    y2[...] = (jnp.dot(lac[...], wa2[...], preferred_element_type=jnp.float32)
               + jnp.dot(blac[...], wb2[...], preferred_element_type=jnp.float32))


def _tc2(lac, blac, params):
    w30 = params["edge_mlp_3"][0][0]
    wa1, wb1, wa2, wb2 = w30[0:128], w30[128:256], w30[256:384], w30[384:512]
    return pl.pallas_call(
        _tc2_body,
        grid=(NC // BR,),
        in_specs=[_rows(BR)] * 2 + [_full((128, 128))] * 4,
        out_specs=[_rows(BR), _rows(BR)],
        out_shape=[jax.ShapeDtypeStruct((NC, 128), jnp.float32),
                   jax.ShapeDtypeStruct((NC, 128), jnp.float32)],
    )(lac, blac, wa1, wb1, wa2, wb2)


def _tc3_body(lvl1h, edge, b30, w31, b31, w32, b32, w0, c0, w1, c1, eps_e, out):
    h = _relu(lvl1h[...] + b30[...])
    h = _relu(jnp.dot(h, w31[...], preferred_element_type=jnp.float32) + b31[...])
    la = jnp.dot(h, w32[...], preferred_element_type=jnp.float32) + b32[...]
    t = (1.0 + eps_e[0, 0]) * edge[...] + la
    h = _relu(jnp.dot(t, w0[...], preferred_element_type=jnp.float32) + c0[...])
    out[...] = jnp.dot(h, w1[...], preferred_element_type=jnp.float32) + c1[...]


def _tc3(lvl1h, edge_attr, params):
    em3 = params["edge_mlp_3"]
    em2 = params["edge_mlp_2"]
    wargs = [em3[0][1], em3[1][0], em3[1][1], em3[2][0], em3[2][1],
             em2[0][0], em2[0][1], em2[1][0], em2[1][1],
             params["eps_edge_1"]]
    wspecs = [_full(w.shape) for w in wargs]
    return pl.pallas_call(
        _tc3_body,
        grid=(E // BR,),
        in_specs=[_rows(BR), _rows(BR)] + wspecs,
        out_specs=_rows(BR),
        out_shape=jax.ShapeDtypeStruct((E, 128), jnp.float32),
    )(lvl1h, edge_attr, *wargs)


_NW = 32
_CG = 256


@functools.partial(jax.jit, static_argnames=("n_chunks",))
def _sc_gather_call(table, idx_pad, n_chunks):
    mesh = plsc.VectorSubcoreMesh(core_axis_name="c", subcore_axis_name="s")
    m_pad = idx_pad.shape[0]
    per_w = m_pad // _NW

    def body(table_hbm, idx_hbm, out_hbm, idx_v, r0, r1, s0, s1, w0, w1):
        wid = lax.axis_index("s") * 2 + lax.axis_index("c")
        base = wid * per_w
        pltpu.sync_copy(idx_hbm.at[pl.ds(base, per_w)], idx_v)
        rvs, gsem, wsem = (r0, r1), (s0, s1), (w0, w1)

        pltpu.async_copy(table_hbm.at[idx_v.at[pl.ds(0, _CG)]], r0, s0)

        def step(k2, carry):
            for b in range(2):
                k = k2 * 2 + b
                nb = (b + 1) % 2
                off = k * _CG
                pltpu.make_async_copy(
                    table_hbm.at[idx_v.at[pl.ds(0, _CG)]], rvs[b],
                    gsem[b]).wait()
                pltpu.async_copy(rvs[b],
                                 out_hbm.at[pl.ds(base + off, _CG)], wsem[b])
                @pl.when(k >= 1)
                def _():
                    pltpu.make_async_copy(
                        rvs[nb], out_hbm.at[pl.ds(0, _CG)], wsem[nb]).wait()

                @pl.when(k < n_chunks - 1)
                def _():
                    pltpu.async_copy(
                        table_hbm.at[idx_v.at[pl.ds(off + _CG, _CG)]],
                        rvs[nb], gsem[nb])
            return carry

        lax.fori_loop(0, n_chunks // 2, step, 0)
        pltpu.make_async_copy(r1, out_hbm.at[pl.ds(0, _CG)], w1).wait()

    f = pl.kernel(
        body,
        out_type=jax.ShapeDtypeStruct((m_pad, 128), jnp.float32),
        mesh=mesh,
        scratch_types=[
            pltpu.VMEM((per_w,), jnp.int32),
            pltpu.VMEM((_CG, 128), jnp.float32),
            pltpu.VMEM((_CG, 128), jnp.float32),
            pltpu.SemaphoreType.DMA, pltpu.SemaphoreType.DMA,
            pltpu.SemaphoreType.DMA, pltpu.SemaphoreType.DMA,
        ],
    )
    return f(table, idx_pad)


def _pad_to(m):
    return ((m + 8191) // 8192) * 8192


def _pad_to_g(m):
    return ((m + 16383) // 16384) * 16384


def _gather(table, idx):
    m_pad = idx.shape[0]
    return _sc_gather_call(table, idx, m_pad // _NW // _CG)


_RSC = 12032
_GF = 128
_SLAB = _RSC // 16


@functools.partial(jax.jit, static_argnames=("npass", "linear"))
def _sc_scat_call(table, src_flat, dst_flat, zeros, npass, linear):
    mesh = plsc.VectorSubcoreMesh(core_axis_name="c", subcore_axis_name="s")
    n_out = npass * 2 * _RSC
    m_pad = dst_flat.shape[0]
    m_slice = m_pad // 16
    nblk = m_slice // 512

    def body(table_hbm, src_hbm, dst_hbm, zeros_hbm, out_hbm,
             dstb, srcb, ss0, ss1, sd0, sd1, rv0, rv1, acc,
             rs0, rs1, as0, as1):
        cid = lax.axis_index("c")
        sid = lax.axis_index("s")
        sbase = sid * m_slice
        sss = (ss0, ss1)
        sds = (sd0, sd1)
        rvs = (rv0, rv1)
        rss = (rs0, rs1)
        ass = (as0, as1)
        trash = jnp.full((16,), _RSC, jnp.int32)

        for p in range(npass):
            base = p * 2 * _RSC + cid * _RSC

            def build(bi, loc):
                for j in range(8):
                    dv = dstb[pl.ds(loc + j * 16, 16)] - base
                    m = (dv >= 0) & (dv < _RSC)
                    sds[bi][pl.ds(j * 16, 16)] = jnp.where(m, dv, _RSC)
                    if not linear:
                        sv = srcb[pl.ds(loc + j * 16, 16)]
                        sss[bi][pl.ds(j * 16, 16)] = jnp.where(m, sv, 0)

            def read(bi, goff):
                if linear:
                    pltpu.async_copy(
                        table_hbm.at[pl.ds(sbase + goff, _GF)],
                        rvs[bi], rss[bi])
                else:
                    pltpu.async_copy(table_hbm.at[sss[bi]], rvs[bi], rss[bi])

            def wait_read(bi):
                if linear:
                    pltpu.make_async_copy(
                        table_hbm.at[pl.ds(0, _GF)], rvs[bi],
                        rss[bi]).wait()
                else:
                    pltpu.make_async_copy(
                        table_hbm.at[sss[bi]], rvs[bi], rss[bi]).wait()

            pltpu.sync_copy(zeros_hbm, rv0)
            for z in range(_SLAB // 128):
                pltpu.sync_copy(rv0, acc.at[pl.ds(sid * _SLAB + z * 128, 128)])
            rem = _SLAB % 128
            if rem:
                pltpu.sync_copy(
                    rv0.at[pl.ds(0, rem)],
                    acc.at[pl.ds(sid * _SLAB + _SLAB - rem, rem)])
            plsc.subcore_barrier()

            for bi in range(2):
                for j in range(8):
                    sds[bi][pl.ds(j * 16, 16)] = trash
                pltpu.async_copy(rvs[bi], acc.at[sds[bi]], ass[bi], add=True)

            def block(nb, carry):
                pltpu.sync_copy(dst_hbm.at[pl.ds(sbase + nb * 512, 512)],
                                dstb)
                if not linear:
                    pltpu.sync_copy(src_hbm.at[pl.ds(sbase + nb * 512, 512)],
                                    srcb)
                for j in range(4):
                    bi = j % 2
                    goff = nb * 512 + j * 128
                    pltpu.make_async_copy(rvs[bi], acc.at[sds[bi]],
                                          ass[bi]).wait()
                    build(bi, j * 128)
                    read(bi, goff)
                    wait_read(bi)
                    pltpu.async_copy(rvs[bi], acc.at[sds[bi]], ass[bi],
                                     add=True)
                return carry

            lax.fori_loop(0, nblk, block, 0)
            for bi in range(2):
                pltpu.make_async_copy(rvs[bi], acc.at[sds[bi]],
                                      ass[bi]).wait()
            plsc.subcore_barrier()
            for z in range(_SLAB // 128):
                pltpu.sync_copy(acc.at[pl.ds(sid * _SLAB + z * 128, 128)],
                                rv0)
                pltpu.sync_copy(rv0, out_hbm.at[
                    pl.ds(base + sid * _SLAB + z * 128, 128)])
            if rem:
                pltpu.sync_copy(
                    acc.at[pl.ds(sid * _SLAB + _SLAB - rem, rem)],
                    rv0.at[pl.ds(0, rem)])
                pltpu.sync_copy(rv0.at[pl.ds(0, rem)], out_hbm.at[
                    pl.ds(base + sid * _SLAB + _SLAB - rem, rem)])

    f = pl.kernel(
        body,
        out_type=jax.ShapeDtypeStruct((n_out, 128), jnp.float32),
        mesh=mesh,
        scratch_types=[
            pltpu.VMEM((512,), jnp.int32),
            pltpu.VMEM((512,), jnp.int32),
            pltpu.VMEM((_GF,), jnp.int32),
            pltpu.VMEM((_GF,), jnp.int32),
            pltpu.VMEM((_GF,), jnp.int32),
            pltpu.VMEM((_GF,), jnp.int32),
            pltpu.VMEM((_GF, 128), jnp.float32),
            pltpu.VMEM((_GF, 128), jnp.float32),
            pltpu.VMEM_SHARED((_RSC + 1, 128), jnp.float32),
            pltpu.SemaphoreType.DMA, pltpu.SemaphoreType.DMA,
            pltpu.SemaphoreType.DMA, pltpu.SemaphoreType.DMA,
        ],
    )
    return f(table, src_flat, dst_flat, zeros)


def _scatter_add(table, src, dst, nrows, linear=False):
    m = dst.shape[0]
    m_pad = _pad_to(m)
    if linear:
        src_p = jnp.zeros((m_pad,), jnp.int32)
        if table.shape[0] < m_pad:
            table = jnp.concatenate(
                [table, jnp.zeros((m_pad - table.shape[0], 128),
                                  jnp.float32)])
    else:
        src_p = jnp.pad(src, (0, m_pad - m))
    dst_p = jnp.pad(dst, (0, m_pad - m), constant_values=-(2 ** 30))
    npass = (nrows + 2 * _RSC - 1) // (2 * _RSC)
    zeros = jnp.zeros((128, 128), jnp.float32)
    out = _sc_scat_call(table, src_p, dst_p, zeros, npass, linear)
    return out


def kernel(edge_attr, cycle_attr, params, cycle_ids,
           e2c_src_1, e2c_dst_1, e2c_src_2, e2c_dst_2,
           c2e_src_1, c2e_dst_1, c2e_src_2, c2e_dst_2):
    e2c1 = _scatter_add(edge_attr, e2c_src_1, e2c_dst_1, NC)
    e2c2 = _scatter_add(edge_attr, e2c_src_2, e2c_dst_2, NC)

    m3 = 3 * NC
    m3_pad = _pad_to(m3)
    table3 = jnp.concatenate([e2c1[:NC], e2c2[:NC], cycle_attr], axis=0)
    dst3 = jnp.concatenate(
        [cycle_ids, cycle_ids + NCYC, cycle_ids + 2 * NCYC])
    segs = _scatter_add(table3, None, dst3, 3 * NCYC, linear=True)

    g3_pad = _pad_to_g(m3)
    gidx = jnp.pad(dst3, (0, g3_pad - m3))
    b3 = _gather(segs, gidx)

    cycle_out, lac = _tc1(e2c1, e2c2, b3, cycle_attr, params)

    slac = _scatter_add(lac, None, cycle_ids, NCYC, linear=True)
    cidp = jnp.pad(cycle_ids, (0, _pad_to_g(NC) - NC))
    blac = _gather(slac, cidp)

    y1, y2 = _tc2(lac, blac[:NC], params)

    ytab = jnp.concatenate([y1, y2], axis=0)
    csrc = jnp.concatenate([c2e_src_1, c2e_src_2 + NC])
    cdst = jnp.concatenate([c2e_dst_1, c2e_dst_2])
    lvl1h = _scatter_add(ytab, csrc, cdst, E)

    edge_out = _tc3(lvl1h, edge_attr, params)
    return (edge_out, cycle_out)


# speedup vs baseline: 25.4330x; 25.4330x over previous
import functools
import jax
import jax.numpy as jnp
from jax import lax
from jax.experimental import pallas as pl
from jax.experimental.pallas import tpu as pltpu
from jax.experimental.pallas import tpu_sc as plsc

E = 160000
NC = 88000
NCYC = 16000

BR = 1000


def _relu(x):
    return jnp.maximum(x, 0.0)


def _full(shape):
    return pl.BlockSpec(shape, lambda i: (0,) * len(shape))


def _rows(br, off=0):
    return pl.BlockSpec((br, 128), lambda i, o=off: (o + i, 0))


def _tc1_body(e2c1, e2c2, b1, b2, ca, bc,
              w20, bb20, w21, bb21, w22, bb22,
              w10, bb10, w11, bb11,
              we0, bbe0, we1, bbe1,
              eps_c,
              cycle_out, lac_out):
    x = jnp.concatenate([e2c2[...], b2[...], e2c1[...], b1[...]], axis=1)
    h = _relu(jnp.dot(x, w20[...], preferred_element_type=jnp.float32) + bb20[...])
    h = _relu(jnp.dot(h, w21[...], preferred_element_type=jnp.float32) + bb21[...])
    lift = jnp.dot(h, w22[...], preferred_element_type=jnp.float32) + bb22[...]

    s = 1.0 + eps_c[0, 0]
    cin = s * jnp.concatenate([ca[...], bc[...]], axis=1) + lift
    h = _relu(jnp.dot(cin, w10[...], preferred_element_type=jnp.float32) + bb10[...])
    cycle_out[...] = jnp.dot(h, w11[...], preferred_element_type=jnp.float32) + bb11[...]

    ein = jnp.concatenate([lift, ca[...]], axis=1)
    h = _relu(jnp.dot(ein, we0[...], preferred_element_type=jnp.float32) + bbe0[...])
    lac_out[...] = jnp.dot(h, we1[...], preferred_element_type=jnp.float32) + bbe1[...]


def _tc1(e2c1, e2c2, b3, ca, params):
    cm2 = params["cycle_mlp_2"]
    cm1 = params["cycle_mlp_1"]
    em1 = params["edge_mlp_1"]
    wargs = [cm2[0][0], cm2[0][1], cm2[1][0], cm2[1][1], cm2[2][0], cm2[2][1],
             cm1[0][0], cm1[0][1], cm1[1][0], cm1[1][1],
             em1[0][0], em1[0][1], em1[1][0], em1[1][1],
             params["eps_cycle_1"]]
    wspecs = [_full(w.shape) for w in wargs]
    nb = NC // BR
    return pl.pallas_call(
        _tc1_body,
        grid=(nb,),
        in_specs=[_rows(BR), _rows(BR), _rows(BR, 0), _rows(BR, nb),
                  _rows(BR), _rows(BR, 2 * nb)] + wspecs,
        out_specs=[_rows(BR), _rows(BR)],
        out_shape=[jax.ShapeDtypeStruct((NC, 128), jnp.float32),
                   jax.ShapeDtypeStruct((NC, 128), jnp.float32)],
    )(e2c1, e2c2, b3, b3, ca, b3, *wargs)


def _tc2_body(lac, blac, wa1, wb1, wa2, wb2, y1, y2):
    y1[...] = (jnp.dot(lac[...], wa1[...], preferred_element_type=jnp.float32)
               + jnp.dot(blac[...], wb1[...], preferred_element_type=jnp.float32))
    y2[...] = (jnp.dot(lac[...], wa2[...], preferred_element_type=jnp.float32)
               + jnp.dot(blac[...], wb2[...], preferred_element_type=jnp.float32))


def _tc2(lac, blac, params):
    w30 = params["edge_mlp_3"][0][0]
    wa1, wb1, wa2, wb2 = w30[0:128], w30[128:256], w30[256:384], w30[384:512]
    return pl.pallas_call(
        _tc2_body,
        grid=(NC // BR,),
        in_specs=[_rows(BR)] * 2 + [_full((128, 128))] * 4,
        out_specs=[_rows(BR), _rows(BR)],
        out_shape=[jax.ShapeDtypeStruct((NC, 128), jnp.float32),
                   jax.ShapeDtypeStruct((NC, 128), jnp.float32)],
    )(lac, blac, wa1, wb1, wa2, wb2)


def _tc3_body(lvl1h, edge, b30, w31, b31, w32, b32, w0, c0, w1, c1, eps_e, out):
    h = _relu(lvl1h[...] + b30[...])
    h = _relu(jnp.dot(h, w31[...], preferred_element_type=jnp.float32) + b31[...])
    la = jnp.dot(h, w32[...], preferred_element_type=jnp.float32) + b32[...]
    t = (1.0 + eps_e[0, 0]) * edge[...] + la
    h = _relu(jnp.dot(t, w0[...], preferred_element_type=jnp.float32) + c0[...])
    out[...] = jnp.dot(h, w1[...], preferred_element_type=jnp.float32) + c1[...]


def _tc3(lvl1h, edge_attr, params):
    em3 = params["edge_mlp_3"]
    em2 = params["edge_mlp_2"]
    wargs = [em3[0][1], em3[1][0], em3[1][1], em3[2][0], em3[2][1],
             em2[0][0], em2[0][1], em2[1][0], em2[1][1],
             params["eps_edge_1"]]
    wspecs = [_full(w.shape) for w in wargs]
    return pl.pallas_call(
        _tc3_body,
        grid=(E // BR,),
        in_specs=[_rows(BR), _rows(BR)] + wspecs,
        out_specs=_rows(BR),
        out_shape=jax.ShapeDtypeStruct((E, 128), jnp.float32),
    )(lvl1h, edge_attr, *wargs)


_NW = 32
_CG = 256


@functools.partial(jax.jit, static_argnames=("n_chunks",))
def _sc_gather_call(table, idx_pad, n_chunks):
    mesh = plsc.VectorSubcoreMesh(core_axis_name="c", subcore_axis_name="s")
    m_pad = idx_pad.shape[0]
    per_w = m_pad // _NW

    def body(table_hbm, idx_hbm, out_hbm, idx_v, rows_v, sem):
        wid = lax.axis_index("s") * 2 + lax.axis_index("c")
        base = wid * per_w
        pltpu.sync_copy(idx_hbm.at[pl.ds(base, per_w)], idx_v)

        def step(k, carry):
            off = k * _CG
            pltpu.async_copy(table_hbm.at[idx_v.at[pl.ds(off, _CG)]],
                             rows_v, sem).wait()
            pltpu.sync_copy(rows_v, out_hbm.at[pl.ds(base + off, _CG)])
            return carry

        lax.fori_loop(0, n_chunks, step, 0)

    f = pl.kernel(
        body,
        out_type=jax.ShapeDtypeStruct((m_pad, 128), jnp.float32),
        mesh=mesh,
        scratch_types=[
            pltpu.VMEM((per_w,), jnp.int32),
            pltpu.VMEM((_CG, 128), jnp.float32),
            pltpu.SemaphoreType.DMA,
        ],
    )
    return f(table, idx_pad)


def _pad_to(m):
    return ((m + 8191) // 8192) * 8192


def _pad_to_g(m):
    return ((m + 16383) // 16384) * 16384


def _gather(table, idx):
    m_pad = idx.shape[0]
    return _sc_gather_call(table, idx, m_pad // _NW // _CG)


_RSC = 11776
_GF = 128
_SLAB = _RSC // 16


@functools.partial(jax.jit, static_argnames=("npass",))
def _sc_scatlin_call(msgs, dst_flat, zeros, npass):
    mesh = plsc.VectorSubcoreMesh(core_axis_name="c", subcore_axis_name="s")
    n_out = npass * 2 * _RSC
    m_pad = dst_flat.shape[0]
    m_slice = m_pad // 16
    ngroups = m_slice // _GF

    def body(msgs_hbm, dst_hbm, zeros_hbm, out_hbm, dst_v, sel_d, rows_v,
             acc, sem):
        cid = lax.axis_index("c")
        sid = lax.axis_index("s")
        pltpu.sync_copy(dst_hbm.at[pl.ds(sid * m_slice, m_slice)], dst_v)

        for p in range(npass):
            base = p * 2 * _RSC + cid * _RSC
            pltpu.sync_copy(zeros_hbm, rows_v)
            for z in range(_SLAB // 128):
                pltpu.sync_copy(rows_v,
                                acc.at[pl.ds(sid * _SLAB + z * 128, 128)])
            rem = _SLAB % 128
            if rem:
                pltpu.sync_copy(
                    rows_v.at[pl.ds(0, rem)],
                    acc.at[pl.ds(sid * _SLAB + _SLAB - rem, rem)])
            plsc.subcore_barrier()

            def group(g, carry):
                goff = g * _GF
                for j in range(8):
                    dv = dst_v[pl.ds(goff + j * 16, 16)] - base
                    m = (dv >= 0) & (dv < _RSC)
                    sel_d[pl.ds(j * 16, 16)] = jnp.where(m, dv, _RSC)
                moff = sid * m_slice + goff
                pltpu.sync_copy(msgs_hbm.at[pl.ds(moff, _GF)], rows_v)
                pltpu.sync_copy(rows_v, acc.at[sel_d], add=True)
                return carry

            lax.fori_loop(0, ngroups, group, 0)
            plsc.subcore_barrier()
            for z in range(_SLAB // 128):
                pltpu.sync_copy(acc.at[pl.ds(sid * _SLAB + z * 128, 128)],
                                rows_v)
                pltpu.sync_copy(rows_v, out_hbm.at[
                    pl.ds(base + sid * _SLAB + z * 128, 128)])
            if rem:
                pltpu.sync_copy(
                    acc.at[pl.ds(sid * _SLAB + _SLAB - rem, rem)],
                    rows_v.at[pl.ds(0, rem)])
                pltpu.sync_copy(rows_v.at[pl.ds(0, rem)], out_hbm.at[
                    pl.ds(base + sid * _SLAB + _SLAB - rem, rem)])

    f = pl.kernel(
        body,
        out_type=jax.ShapeDtypeStruct((n_out, 128), jnp.float32),
        mesh=mesh,
        scratch_types=[
            pltpu.VMEM((m_slice,), jnp.int32),
            pltpu.VMEM((_GF,), jnp.int32),
            pltpu.VMEM((_GF, 128), jnp.float32),
            pltpu.VMEM_SHARED((_RSC + 1, 128), jnp.float32),
            pltpu.SemaphoreType.DMA,
        ],
    )
    return f(msgs, dst_flat, zeros)


def _scatter_add(table, src, dst, nrows, linear=False):
    m = dst.shape[0]
    m_pad = _pad_to(m) if linear else _pad_to_g(m)
    if linear:
        msgs = table
        if msgs.shape[0] < m_pad:
            msgs = jnp.concatenate(
                [msgs, jnp.zeros((m_pad - msgs.shape[0], 128),
                                 jnp.float32)])
    else:
        idx_pad = jnp.pad(src, (0, m_pad - m))
        msgs = _gather(table, idx_pad)
    dst_p = jnp.pad(dst, (0, m_pad - m), constant_values=-(2 ** 30))
    npass = (nrows + 2 * _RSC - 1) // (2 * _RSC)
    zeros = jnp.zeros((128, 128), jnp.float32)
    return _sc_scatlin_call(msgs, dst_p, zeros, npass)


def kernel(edge_attr, cycle_attr, params, cycle_ids,
           e2c_src_1, e2c_dst_1, e2c_src_2, e2c_dst_2,
           c2e_src_1, c2e_dst_1, c2e_src_2, c2e_dst_2):
    e2c1 = _scatter_add(edge_attr, e2c_src_1, e2c_dst_1, NC)
    e2c2 = _scatter_add(edge_attr, e2c_src_2, e2c_dst_2, NC)

    m3 = 3 * NC
    m3_pad = _pad_to(m3)
    table3 = jnp.concatenate([e2c1[:NC], e2c2[:NC], cycle_attr], axis=0)
    dst3 = jnp.concatenate(
        [cycle_ids, cycle_ids + NCYC, cycle_ids + 2 * NCYC])
    segs = _scatter_add(table3, None, dst3, 3 * NCYC, linear=True)

    g3_pad = _pad_to_g(m3)
    gidx = jnp.pad(dst3, (0, g3_pad - m3))
    b3 = _gather(segs, gidx)

    cycle_out, lac = _tc1(e2c1, e2c2, b3, cycle_attr, params)

    slac = _scatter_add(lac, None, cycle_ids, NCYC, linear=True)
    cidp = jnp.pad(cycle_ids, (0, _pad_to_g(NC) - NC))
    blac = _gather(slac, cidp)

    y1, y2 = _tc2(lac, blac[:NC], params)

    ytab = jnp.concatenate([y1, y2], axis=0)
    csrc = jnp.concatenate([c2e_src_1, c2e_src_2 + NC])
    cdst = jnp.concatenate([c2e_dst_1, c2e_dst_2])
    lvl1h = _scatter_add(ytab, csrc, cdst, E)

    edge_out = _tc3(lvl1h, edge_attr, params)
    return (edge_out, cycle_out)


# speedup vs baseline: 25.5365x; 1.0041x over previous
import functools
import jax
import jax.numpy as jnp
from jax import lax
from jax.experimental import pallas as pl
from jax.experimental.pallas import tpu as pltpu
from jax.experimental.pallas import tpu_sc as plsc

E = 160000
NC = 88000
NCYC = 16000

BR = 1000


def _relu(x):
    return jnp.maximum(x, 0.0)


def _full(shape):
    return pl.BlockSpec(shape, lambda i: (0,) * len(shape))


def _rows(br, off=0):
    return pl.BlockSpec((br, 128), lambda i, o=off: (o + i, 0))


def _tc1_body(e2c1, e2c2, b1, b2, ca, bc,
              w20, bb20, w21, bb21, w22, bb22,
              w10, bb10, w11, bb11,
              we0, bbe0, we1, bbe1,
              eps_c,
              cycle_out, lac_out):
    x = jnp.concatenate([e2c2[...], b2[...], e2c1[...], b1[...]], axis=1)
    h = _relu(jnp.dot(x, w20[...], preferred_element_type=jnp.float32) + bb20[...])
    h = _relu(jnp.dot(h, w21[...], preferred_element_type=jnp.float32) + bb21[...])
    lift = jnp.dot(h, w22[...], preferred_element_type=jnp.float32) + bb22[...]

    s = 1.0 + eps_c[0, 0]
    cin = s * jnp.concatenate([ca[...], bc[...]], axis=1) + lift
    h = _relu(jnp.dot(cin, w10[...], preferred_element_type=jnp.float32) + bb10[...])
    cycle_out[...] = jnp.dot(h, w11[...], preferred_element_type=jnp.float32) + bb11[...]

    ein = jnp.concatenate([lift, ca[...]], axis=1)
    h = _relu(jnp.dot(ein, we0[...], preferred_element_type=jnp.float32) + bbe0[...])
    lac_out[...] = jnp.dot(h, we1[...], preferred_element_type=jnp.float32) + bbe1[...]


def _tc1(e2c1, e2c2, b3, ca, params):
    cm2 = params["cycle_mlp_2"]
    cm1 = params["cycle_mlp_1"]
    em1 = params["edge_mlp_1"]
    wargs = [cm2[0][0], cm2[0][1], cm2[1][0], cm2[1][1], cm2[2][0], cm2[2][1],
             cm1[0][0], cm1[0][1], cm1[1][0], cm1[1][1],
             em1[0][0], em1[0][1], em1[1][0], em1[1][1],
             params["eps_cycle_1"]]
    wspecs = [_full(w.shape) for w in wargs]
    nb = NC // BR
    return pl.pallas_call(
        _tc1_body,
        grid=(nb,),
        in_specs=[_rows(BR), _rows(BR), _rows(BR, 0), _rows(BR, nb),
                  _rows(BR), _rows(BR, 2 * nb)] + wspecs,
        out_specs=[_rows(BR), _rows(BR)],
        out_shape=[jax.ShapeDtypeStruct((NC, 128), jnp.float32),
                   jax.ShapeDtypeStruct((NC, 128), jnp.float32)],
    )(e2c1, e2c2, b3, b3, ca, b3, *wargs)


def _tc2_body(lac, blac, wa1, wb1, wa2, wb2, y1, y2):
    y1[...] = (jnp.dot(lac[...], wa1[...], preferred_element_type=jnp.float32)
               + jnp.dot(blac[...], wb1[...], preferred_element_type=jnp.float32))
    y2[...] = (jnp.dot(lac[...], wa2[...], preferred_element_type=jnp.float32)
               + jnp.dot(blac[...], wb2[...], preferred_element_type=jnp.float32))


def _tc2(lac, blac, params):
    w30 = params["edge_mlp_3"][0][0]
    wa1, wb1, wa2, wb2 = w30[0:128], w30[128:256], w30[256:384], w30[384:512]
    return pl.pallas_call(
        _tc2_body,
        grid=(NC // BR,),
        in_specs=[_rows(BR)] * 2 + [_full((128, 128))] * 4,
        out_specs=[_rows(BR), _rows(BR)],
        out_shape=[jax.ShapeDtypeStruct((NC, 128), jnp.float32),
                   jax.ShapeDtypeStruct((NC, 128), jnp.float32)],
    )(lac, blac, wa1, wb1, wa2, wb2)


def _tc3_body(lvl1h, edge, b30, w31, b31, w32, b32, w0, c0, w1, c1, eps_e, out):
    h = _relu(lvl1h[...] + b30[...])
    h = _relu(jnp.dot(h, w31[...], preferred_element_type=jnp.float32) + b31[...])
    la = jnp.dot(h, w32[...], preferred_element_type=jnp.float32) + b32[...]
    t = (1.0 + eps_e[0, 0]) * edge[...] + la
    h = _relu(jnp.dot(t, w0[...], preferred_element_type=jnp.float32) + c0[...])
    out[...] = jnp.dot(h, w1[...], preferred_element_type=jnp.float32) + c1[...]


def _tc3(lvl1h, edge_attr, params):
    em3 = params["edge_mlp_3"]
    em2 = params["edge_mlp_2"]
    wargs = [em3[0][1], em3[1][0], em3[1][1], em3[2][0], em3[2][1],
             em2[0][0], em2[0][1], em2[1][0], em2[1][1],
             params["eps_edge_1"]]
    wspecs = [_full(w.shape) for w in wargs]
    return pl.pallas_call(
        _tc3_body,
        grid=(E // BR,),
        in_specs=[_rows(BR), _rows(BR)] + wspecs,
        out_specs=_rows(BR),
        out_shape=jax.ShapeDtypeStruct((E, 128), jnp.float32),
    )(lvl1h, edge_attr, *wargs)


_NW = 32
_CG = 512


@functools.partial(jax.jit, static_argnames=("n_chunks",))
def _sc_gather_call(table, idx_pad, n_chunks):
    mesh = plsc.VectorSubcoreMesh(core_axis_name="c", subcore_axis_name="s")
    m_pad = idx_pad.shape[0]
    per_w = m_pad // _NW

    def body(table_hbm, idx_hbm, out_hbm, idx_v, rows_v, sem):
        wid = lax.axis_index("s") * 2 + lax.axis_index("c")
        base = wid * per_w
        pltpu.sync_copy(idx_hbm.at[pl.ds(base, per_w)], idx_v)

        def step(k, carry):
            off = k * _CG
            pltpu.async_copy(table_hbm.at[idx_v.at[pl.ds(off, _CG)]],
                             rows_v, sem).wait()
            pltpu.sync_copy(rows_v, out_hbm.at[pl.ds(base + off, _CG)])
            return carry

        lax.fori_loop(0, n_chunks, step, 0)

    f = pl.kernel(
        body,
        out_type=jax.ShapeDtypeStruct((m_pad, 128), jnp.float32),
        mesh=mesh,
        scratch_types=[
            pltpu.VMEM((per_w,), jnp.int32),
            pltpu.VMEM((_CG, 128), jnp.float32),
            pltpu.SemaphoreType.DMA,
        ],
    )
    return f(table, idx_pad)


def _pad_to(m):
    return ((m + 8191) // 8192) * 8192


def _pad_to_g(m):
    return ((m + 16383) // 16384) * 16384


def _gather(table, idx):
    m_pad = idx.shape[0]
    return _sc_gather_call(table, idx, m_pad // _NW // _CG)


_RSC = 11776
_GF = 128
_SLAB = _RSC // 16


@functools.partial(jax.jit, static_argnames=("npass",))
def _sc_scatlin_call(msgs, dst_flat, zeros, npass):
    mesh = plsc.VectorSubcoreMesh(core_axis_name="c", subcore_axis_name="s")
    n_out = npass * 2 * _RSC
    m_pad = dst_flat.shape[0]
    m_slice = m_pad // 16
    ngroups = m_slice // _GF

    def body(msgs_hbm, dst_hbm, zeros_hbm, out_hbm, dst_v, sel_d, rows_v,
             acc, sem):
        cid = lax.axis_index("c")
        sid = lax.axis_index("s")
        pltpu.sync_copy(dst_hbm.at[pl.ds(sid * m_slice, m_slice)], dst_v)

        for p in range(npass):
            base = p * 2 * _RSC + cid * _RSC
            pltpu.sync_copy(zeros_hbm, rows_v)
            for z in range(_SLAB // 128):
                pltpu.sync_copy(rows_v,
                                acc.at[pl.ds(sid * _SLAB + z * 128, 128)])
            rem = _SLAB % 128
            if rem:
                pltpu.sync_copy(
                    rows_v.at[pl.ds(0, rem)],
                    acc.at[pl.ds(sid * _SLAB + _SLAB - rem, rem)])
            plsc.subcore_barrier()

            def group(g, carry):
                goff = g * _GF
                for j in range(8):
                    dv = dst_v[pl.ds(goff + j * 16, 16)] - base
                    m = (dv >= 0) & (dv < _RSC)
                    sel_d[pl.ds(j * 16, 16)] = jnp.where(m, dv, _RSC)
                moff = sid * m_slice + goff
                pltpu.sync_copy(msgs_hbm.at[pl.ds(moff, _GF)], rows_v)
                pltpu.sync_copy(rows_v, acc.at[sel_d], add=True)
                return carry

            lax.fori_loop(0, ngroups, group, 0)
            plsc.subcore_barrier()
            for z in range(_SLAB // 128):
                pltpu.sync_copy(acc.at[pl.ds(sid * _SLAB + z * 128, 128)],
                                rows_v)
                pltpu.sync_copy(rows_v, out_hbm.at[
                    pl.ds(base + sid * _SLAB + z * 128, 128)])
            if rem:
                pltpu.sync_copy(
                    acc.at[pl.ds(sid * _SLAB + _SLAB - rem, rem)],
                    rows_v.at[pl.ds(0, rem)])
                pltpu.sync_copy(rows_v.at[pl.ds(0, rem)], out_hbm.at[
                    pl.ds(base + sid * _SLAB + _SLAB - rem, rem)])

    f = pl.kernel(
        body,
        out_type=jax.ShapeDtypeStruct((n_out, 128), jnp.float32),
        mesh=mesh,
        scratch_types=[
            pltpu.VMEM((m_slice,), jnp.int32),
            pltpu.VMEM((_GF,), jnp.int32),
            pltpu.VMEM((_GF, 128), jnp.float32),
            pltpu.VMEM_SHARED((_RSC + 1, 128), jnp.float32),
            pltpu.SemaphoreType.DMA,
        ],
    )
    return f(msgs, dst_flat, zeros)


def _scatter_add(table, src, dst, nrows, linear=False):
    m = dst.shape[0]
    m_pad = _pad_to(m) if linear else _pad_to_g(m)
    if linear:
        msgs = table
        if msgs.shape[0] < m_pad:
            msgs = jnp.concatenate(
                [msgs, jnp.zeros((m_pad - msgs.shape[0], 128),
                                 jnp.float32)])
    else:
        idx_pad = jnp.pad(src, (0, m_pad - m))
        msgs = _gather(table, idx_pad)
    dst_p = jnp.pad(dst, (0, m_pad - m), constant_values=-(2 ** 30))
    npass = (nrows + 2 * _RSC - 1) // (2 * _RSC)
    zeros = jnp.zeros((128, 128), jnp.float32)
    return _sc_scatlin_call(msgs, dst_p, zeros, npass)


def kernel(edge_attr, cycle_attr, params, cycle_ids,
           e2c_src_1, e2c_dst_1, e2c_src_2, e2c_dst_2,
           c2e_src_1, c2e_dst_1, c2e_src_2, c2e_dst_2):
    e2c1 = _scatter_add(edge_attr, e2c_src_1, e2c_dst_1, NC)
    e2c2 = _scatter_add(edge_attr, e2c_src_2, e2c_dst_2, NC)

    m3 = 3 * NC
    m3_pad = _pad_to(m3)
    table3 = jnp.concatenate([e2c1[:NC], e2c2[:NC], cycle_attr], axis=0)
    dst3 = jnp.concatenate(
        [cycle_ids, cycle_ids + NCYC, cycle_ids + 2 * NCYC])
    segs = _scatter_add(table3, None, dst3, 3 * NCYC, linear=True)

    g3_pad = _pad_to_g(m3)
    gidx = jnp.pad(dst3, (0, g3_pad - m3))
    b3 = _gather(segs, gidx)

    cycle_out, lac = _tc1(e2c1, e2c2, b3, cycle_attr, params)

    slac = _scatter_add(lac, None, cycle_ids, NCYC, linear=True)
    cidp = jnp.pad(cycle_ids, (0, _pad_to_g(NC) - NC))
    blac = _gather(slac, cidp)

    y1, y2 = _tc2(lac, blac[:NC], params)

    ytab = jnp.concatenate([y1, y2], axis=0)
    csrc = jnp.concatenate([c2e_src_1, c2e_src_2 + NC])
    cdst = jnp.concatenate([c2e_dst_1, c2e_dst_2])
    lvl1h = _scatter_add(ytab, csrc, cdst, E)

    edge_out = _tc3(lvl1h, edge_attr, params)
    return (edge_out, cycle_out)


# speedup vs baseline: 29.6270x; 1.1602x over previous
import functools
import jax
import jax.numpy as jnp
from jax import lax
from jax.experimental import pallas as pl
from jax.experimental.pallas import tpu as pltpu
from jax.experimental.pallas import tpu_sc as plsc

E = 160000
NC = 88000
NCYC = 16000

BR = 1000


def _relu(x):
    return jnp.maximum(x, 0.0)


def _full(shape):
    return pl.BlockSpec(shape, lambda i: (0,) * len(shape))


def _rows(br, off=0):
    return pl.BlockSpec((br, 128), lambda i, o=off: (o + i, 0))


def _tc1_body(e2c1, e2c2, b1, b2, ca, bc,
              w20, bb20, w21, bb21, w22, bb22,
              w10, bb10, w11, bb11,
              we0, bbe0, we1, bbe1,
              eps_c,
              cycle_out, lac_out):
    x = jnp.concatenate([e2c2[...], b2[...], e2c1[...], b1[...]], axis=1)
    h = _relu(jnp.dot(x, w20[...], preferred_element_type=jnp.float32) + bb20[...])
    h = _relu(jnp.dot(h, w21[...], preferred_element_type=jnp.float32) + bb21[...])
    lift = jnp.dot(h, w22[...], preferred_element_type=jnp.float32) + bb22[...]

    s = 1.0 + eps_c[0, 0]
    cin = s * jnp.concatenate([ca[...], bc[...]], axis=1) + lift
    h = _relu(jnp.dot(cin, w10[...], preferred_element_type=jnp.float32) + bb10[...])
    cycle_out[...] = jnp.dot(h, w11[...], preferred_element_type=jnp.float32) + bb11[...]

    ein = jnp.concatenate([lift, ca[...]], axis=1)
    h = _relu(jnp.dot(ein, we0[...], preferred_element_type=jnp.float32) + bbe0[...])
    lac_out[...] = jnp.dot(h, we1[...], preferred_element_type=jnp.float32) + bbe1[...]


def _tc1(e2c1, e2c2, b3, ca, params):
    cm2 = params["cycle_mlp_2"]
    cm1 = params["cycle_mlp_1"]
    em1 = params["edge_mlp_1"]
    wargs = [cm2[0][0], cm2[0][1], cm2[1][0], cm2[1][1], cm2[2][0], cm2[2][1],
             cm1[0][0], cm1[0][1], cm1[1][0], cm1[1][1],
             em1[0][0], em1[0][1], em1[1][0], em1[1][1],
             params["eps_cycle_1"]]
    wspecs = [_full(w.shape) for w in wargs]
    b1, b2, bc = b3[:NC], b3[NC:2 * NC], b3[2 * NC:3 * NC]
    return pl.pallas_call(
        _tc1_body,
        grid=(NC // BR,),
        in_specs=[_rows(BR)] * 6 + wspecs,
        out_specs=[_rows(BR), _rows(BR)],
        out_shape=[jax.ShapeDtypeStruct((NC, 128), jnp.float32),
                   jax.ShapeDtypeStruct((NC, 128), jnp.float32)],
    )(e2c1[:NC], e2c2[:NC], b1, b2, ca, bc, *wargs)


def _tc2_body(lac, blac, wa1, wb1, wa2, wb2, y1, y2):
    y1[...] = (jnp.dot(lac[...], wa1[...], preferred_element_type=jnp.float32)
               + jnp.dot(blac[...], wb1[...], preferred_element_type=jnp.float32))
    y2[...] = (jnp.dot(lac[...], wa2[...], preferred_element_type=jnp.float32)
               + jnp.dot(blac[...], wb2[...], preferred_element_type=jnp.float32))


def _tc2(lac, blac, params):
    w30 = params["edge_mlp_3"][0][0]
    wa1, wb1, wa2, wb2 = w30[0:128], w30[128:256], w30[256:384], w30[384:512]
    return pl.pallas_call(
        _tc2_body,
        grid=(NC // BR,),
        in_specs=[_rows(BR)] * 2 + [_full((128, 128))] * 4,
        out_specs=[_rows(BR), _rows(BR)],
        out_shape=[jax.ShapeDtypeStruct((NC, 128), jnp.float32),
                   jax.ShapeDtypeStruct((NC, 128), jnp.float32)],
    )(lac, blac, wa1, wb1, wa2, wb2)


def _tc3_body(lvl1h, edge, b30, w31, b31, w32, b32, w0, c0, w1, c1, eps_e, out):
    h = _relu(lvl1h[...] + b30[...])
    h = _relu(jnp.dot(h, w31[...], preferred_element_type=jnp.float32) + b31[...])
    la = jnp.dot(h, w32[...], preferred_element_type=jnp.float32) + b32[...]
    t = (1.0 + eps_e[0, 0]) * edge[...] + la
    h = _relu(jnp.dot(t, w0[...], preferred_element_type=jnp.float32) + c0[...])
    out[...] = jnp.dot(h, w1[...], preferred_element_type=jnp.float32) + c1[...]


def _tc3(lvl1h, edge_attr, params):
    em3 = params["edge_mlp_3"]
    em2 = params["edge_mlp_2"]
    wargs = [em3[0][1], em3[1][0], em3[1][1], em3[2][0], em3[2][1],
             em2[0][0], em2[0][1], em2[1][0], em2[1][1],
             params["eps_edge_1"]]
    wspecs = [_full(w.shape) for w in wargs]
    return pl.pallas_call(
        _tc3_body,
        grid=(E // BR,),
        in_specs=[_rows(BR), _rows(BR)] + wspecs,
        out_specs=_rows(BR),
        out_shape=jax.ShapeDtypeStruct((E, 128), jnp.float32),
    )(lvl1h, edge_attr, *wargs)


_NW = 32
_CG = 256


@functools.partial(jax.jit, static_argnames=("n_chunks",))
def _sc_gather_call(table, idx_pad, n_chunks):
    mesh = plsc.VectorSubcoreMesh(core_axis_name="c", subcore_axis_name="s")
    m_pad = idx_pad.shape[0]
    per_w = m_pad // _NW

    def body(table_hbm, idx_hbm, out_hbm, idx_v, rows_v, sem):
        wid = lax.axis_index("s") * 2 + lax.axis_index("c")
        base = wid * per_w
        pltpu.sync_copy(idx_hbm.at[pl.ds(base, per_w)], idx_v)

        def step(k, carry):
            off = k * _CG
            pltpu.async_copy(table_hbm.at[idx_v.at[pl.ds(off, _CG)]],
                             rows_v, sem).wait()
            pltpu.sync_copy(rows_v, out_hbm.at[pl.ds(base + off, _CG)])
            return carry

        lax.fori_loop(0, n_chunks, step, 0)

    f = pl.kernel(
        body,
        out_type=jax.ShapeDtypeStruct((m_pad, 128), jnp.float32),
        mesh=mesh,
        scratch_types=[
            pltpu.VMEM((per_w,), jnp.int32),
            pltpu.VMEM((_CG, 128), jnp.float32),
            pltpu.SemaphoreType.DMA,
        ],
    )
    return f(table, idx_pad)


def _pad_to(m):
    return ((m + 8191) // 8192) * 8192


def _pad_to_g(m):
    return ((m + 8191) // 8192) * 8192


def _gather(table, idx):
    m_pad = idx.shape[0]
    return _sc_gather_call(table, idx, m_pad // _NW // _CG)


_RSC = 11776
_GF = 128
_SLAB = _RSC // 16


@functools.partial(jax.jit, static_argnames=("npass",))
def _sc_scatlin_call(msgs, dst_flat, zeros, npass):
    mesh = plsc.VectorSubcoreMesh(core_axis_name="c", subcore_axis_name="s")
    n_out = npass * 2 * _RSC
    m_pad = dst_flat.shape[0]
    m_slice = m_pad // 16
    ngroups = m_slice // _GF

    def body(msgs_hbm, dst_hbm, zeros_hbm, out_hbm, dst_v, sel_d, rows_v,
             acc, sem):
        cid = lax.axis_index("c")
        sid = lax.axis_index("s")
        pltpu.sync_copy(dst_hbm.at[pl.ds(sid * m_slice, m_slice)], dst_v)

        for p in range(npass):
            base = p * 2 * _RSC + cid * _RSC
            pltpu.sync_copy(zeros_hbm, rows_v)
            for z in range(_SLAB // 128):
                pltpu.sync_copy(rows_v,
                                acc.at[pl.ds(sid * _SLAB + z * 128, 128)])
            rem = _SLAB % 128
            if rem:
                pltpu.sync_copy(
                    rows_v.at[pl.ds(0, rem)],
                    acc.at[pl.ds(sid * _SLAB + _SLAB - rem, rem)])
            plsc.subcore_barrier()

            def group(g, carry):
                goff = g * _GF
                for j in range(8):
                    dv = dst_v[pl.ds(goff + j * 16, 16)] - base
                    m = (dv >= 0) & (dv < _RSC)
                    sel_d[pl.ds(j * 16, 16)] = jnp.where(m, dv, _RSC)
                moff = sid * m_slice + goff
                pltpu.sync_copy(msgs_hbm.at[pl.ds(moff, _GF)], rows_v)
                pltpu.sync_copy(rows_v, acc.at[sel_d], add=True)
                return carry

            lax.fori_loop(0, ngroups, group, 0)
            plsc.subcore_barrier()
            for z in range(_SLAB // 128):
                pltpu.sync_copy(acc.at[pl.ds(sid * _SLAB + z * 128, 128)],
                                rows_v)
                pltpu.sync_copy(rows_v, out_hbm.at[
                    pl.ds(base + sid * _SLAB + z * 128, 128)])
            if rem:
                pltpu.sync_copy(
                    acc.at[pl.ds(sid * _SLAB + _SLAB - rem, rem)],
                    rows_v.at[pl.ds(0, rem)])
                pltpu.sync_copy(rows_v.at[pl.ds(0, rem)], out_hbm.at[
                    pl.ds(base + sid * _SLAB + _SLAB - rem, rem)])

    f = pl.kernel(
        body,
        out_type=jax.ShapeDtypeStruct((n_out, 128), jnp.float32),
        mesh=mesh,
        scratch_types=[
            pltpu.VMEM((m_slice,), jnp.int32),
            pltpu.VMEM((_GF,), jnp.int32),
            pltpu.VMEM((_GF, 128), jnp.float32),
            pltpu.VMEM_SHARED((_RSC + 1, 128), jnp.float32),
            pltpu.SemaphoreType.DMA,
        ],
    )
    return f(msgs, dst_flat, zeros)


def _scatter_add(table, src, dst, nrows, linear=False):
    m = dst.shape[0]
    m_pad = _pad_to(m) if linear else _pad_to_g(m)
    if linear:
        msgs = table
        if msgs.shape[0] < m_pad:
            msgs = jnp.concatenate(
                [msgs, jnp.zeros((m_pad - msgs.shape[0], 128),
                                 jnp.float32)])
    else:
        idx_pad = jnp.pad(src, (0, m_pad - m))
        msgs = _gather(table, idx_pad)
    dst_p = jnp.pad(dst, (0, m_pad - m), constant_values=-(2 ** 30))
    npass = (nrows + 2 * _RSC - 1) // (2 * _RSC)
    zeros = jnp.zeros((128, 128), jnp.float32)
    return _sc_scatlin_call(msgs, dst_p, zeros, npass)


def kernel(edge_attr, cycle_attr, params, cycle_ids,
           e2c_src_1, e2c_dst_1, e2c_src_2, e2c_dst_2,
           c2e_src_1, c2e_dst_1, c2e_src_2, c2e_dst_2):
    e2c1 = _scatter_add(edge_attr, e2c_src_1, e2c_dst_1, NC)
    e2c2 = _scatter_add(edge_attr, e2c_src_2, e2c_dst_2, NC)

    m3 = 3 * NC
    m3_pad = _pad_to(m3)
    table3 = jnp.concatenate([e2c1[:NC], e2c2[:NC], cycle_attr], axis=0)
    dst3 = jnp.concatenate(
        [cycle_ids, cycle_ids + NCYC, cycle_ids + 2 * NCYC])
    segs = _scatter_add(table3, None, dst3, 3 * NCYC, linear=True)

    g3_pad = _pad_to_g(m3)
    gidx = jnp.pad(dst3, (0, g3_pad - m3))
    b3 = _gather(segs, gidx)

    cycle_out, lac = _tc1(e2c1, e2c2, b3, cycle_attr, params)

    slac = _scatter_add(lac, None, cycle_ids, NCYC, linear=True)
    cidp = jnp.pad(cycle_ids, (0, _pad_to_g(NC) - NC))
    blac = _gather(slac, cidp)

    y1, y2 = _tc2(lac, blac[:NC], params)

    ytab = jnp.concatenate([y1, y2], axis=0)
    csrc = jnp.concatenate([c2e_src_1, c2e_src_2 + NC])
    cdst = jnp.concatenate([c2e_dst_1, c2e_dst_2])
    lvl1h = _scatter_add(ytab, csrc, cdst, E)

    edge_out = _tc3(lvl1h, edge_attr, params)
    return (edge_out, cycle_out)
